# baseline (speedup 1.0000x reference)
import functools

import jax
import jax.numpy as jnp
from jax import lax
from jax.experimental import pallas as pl
from jax.experimental.pallas import tpu as pltpu
from jax.experimental.pallas import tpu_sc as plsc

_F32 = jnp.float32
_I32 = jnp.int32


_BM = 512


def _interleave_mat(bm):
    col = jax.lax.broadcasted_iota(_I32, (bm // 2, bm), 1)
    row = jax.lax.broadcasted_iota(_I32, (bm // 2, bm), 0)
    return (col // 2 == row).astype(_F32)


def _matmul1(prevs, enc, wts, relu_prev):
    c_e, n = enc.shape
    c_pr = prevs[0].shape[0]
    np_ = len(prevs)
    nw = len(wts)
    rmat = _interleave_mat(_BM)

    def body(*refs):
        prefs = refs[:np_]
        enc_ref = refs[np_]
        rref = refs[np_ + 1]
        wrefs = refs[np_ + 2:np_ + 2 + 3 * nw]
        orefs = refs[np_ + 2 + 3 * nw:]
        p = prefs[0][...]
        for pr in prefs[1:]:
            p = p + pr[...]
        if relu_prev:
            p = jnp.maximum(p, 0.0)
        e = enc_ref[...]
        r = rref[...]
        for j in range(nw):
            wt = wrefs[3 * j][...]
            wb = wrefs[3 * j + 1][...]
            bb = wrefs[3 * j + 2][...]
            y = jnp.dot(jnp.dot(wt, p, preferred_element_type=_F32), r,
                        preferred_element_type=_F32)
            orefs[j][...] = y + jnp.dot(wb, e, preferred_element_type=_F32) + bb

    in_specs = [pl.BlockSpec((c_pr, _BM // 2), lambda i: (0, i))] * np_
    in_specs.append(pl.BlockSpec((c_e, _BM), lambda i: (0, i)))
    in_specs.append(pl.BlockSpec((_BM // 2, _BM), lambda i: (0, 0)))
    out_specs = []
    out_shape = []
    for (wt, wb, bb) in wts:
        c = wt.shape[0]
        in_specs.append(pl.BlockSpec(wt.shape, lambda i: (0, 0)))
        in_specs.append(pl.BlockSpec(wb.shape, lambda i: (0, 0)))
        in_specs.append(pl.BlockSpec(bb.shape, lambda i: (0, 0)))
        out_specs.append(pl.BlockSpec((c, _BM), lambda i: (0, i)))
        out_shape.append(jax.ShapeDtypeStruct((c, n), _F32))

    args = list(prevs) + [enc, rmat]
    for (wt, wb, bb) in wts:
        args += [wt, wb, bb]
    return pl.pallas_call(
        body, grid=(n // _BM,), in_specs=in_specs, out_specs=out_specs,
        out_shape=out_shape)(*args)


def _matmul2(ha, hb, s, w0t, w1t, w2t, b2):
    c, n = ha.shape

    def body(ha_ref, hb_ref, s_ref, w0_ref, w1_ref, w2_ref, b2_ref,
             o0, o1, o2):
        h = jnp.maximum(ha_ref[...] + hb_ref[...], 0.0)
        o0[...] = (jnp.dot(w0_ref[...], h, preferred_element_type=_F32)
                   + b2_ref[...] + s_ref[...])
        o1[...] = jnp.dot(w1_ref[...], h, preferred_element_type=_F32)
        o2[...] = jnp.dot(w2_ref[...], h, preferred_element_type=_F32)

    bspec = pl.BlockSpec((c, _BM), lambda i: (0, i))
    wspec = pl.BlockSpec((c, c), lambda i: (0, 0))
    return pl.pallas_call(
        body, grid=(n // _BM,),
        in_specs=[bspec, bspec, bspec, wspec, wspec, wspec,
                  pl.BlockSpec((c, 1), lambda i: (0, 0))],
        out_specs=[bspec, bspec, bspec],
        out_shape=[jax.ShapeDtypeStruct((c, n), _F32)] * 3,
    )(ha, hb, s, w0t, w1t, w2t, b2)


def _comb2(a, b):
    c, n = a.shape

    def body(a_ref, b_ref, o_ref):
        o_ref[...] = a_ref[...] + b_ref[...]

    bspec = pl.BlockSpec((c, _BM), lambda i: (0, i))
    return pl.pallas_call(
        body, grid=(n // _BM,), in_specs=[bspec, bspec], out_specs=bspec,
        out_shape=jax.ShapeDtypeStruct((c, n), _F32))(a, b)


def _rsqrt_deg(hs_p, hd_p):
    n = hs_p.shape[1]

    def body(hs_ref, hd_ref, rs_ref, ri_ref):
        ds = jnp.maximum(jnp.sum(hs_ref[...], axis=0), 1.0)
        di = jnp.maximum(jnp.sum(hd_ref[...], axis=0), 1.0)
        rs_ref[...] = lax.rsqrt(ds)
        ri_ref[...] = lax.rsqrt(di)

    return pl.pallas_call(
        body,
        in_specs=[pl.BlockSpec((16, n), lambda: (0, 0))] * 2,
        out_specs=[pl.BlockSpec((n,), lambda: (0,))] * 2,
        out_shape=[jax.ShapeDtypeStruct((n,), _F32)] * 2,
    )(hs_p, hd_p)


def _final_head(oa, ob, wc_t):
    n = oa.shape[1]

    def body(oa_ref, ob_ref, w_ref, out_ref):
        o = jnp.maximum(oa_ref[...] + ob_ref[...], 0.0)
        pooled = jnp.mean(o, axis=1, keepdims=True)
        logits = jnp.dot(w_ref[...], pooled, preferred_element_type=_F32)
        m = jnp.max(logits, axis=0, keepdims=True)
        z = logits - m
        lse = jnp.log(jnp.sum(jnp.exp(z), axis=0, keepdims=True))
        out_ref[...] = z - lse

    return pl.pallas_call(
        body,
        in_specs=[pl.BlockSpec((16, n), lambda: (0, 0)),
                  pl.BlockSpec((16, n), lambda: (0, 0)),
                  pl.BlockSpec((32, 16), lambda: (0, 0))],
        out_specs=pl.BlockSpec((32, 1), lambda: (0, 0)),
        out_shape=jax.ShapeDtypeStruct((32, 1), _F32),
    )(oa, ob, wc_t)


_MESH = dict(core_axis_name="c", subcore_axis_name="s")
_SC_PARAMS = dict(
    mesh=plsc.VectorSubcoreMesh(**_MESH),
    compiler_params=pltpu.CompilerParams(needs_layout_passes=False),
)


def _sc_degree_hist(src, dst, n):
    e = src.shape[0]
    ept = e // 16
    win = 2560
    nwin = ept // win

    @functools.partial(
        pl.kernel,
        out_type=[jax.ShapeDtypeStruct((16, n), _F32)] * 2,
        scratch_types=[
            pltpu.VMEM((n,), _F32),
            pltpu.VMEM((win,), _I32),
        ],
        **_SC_PARAMS,
    )
    def k(src_hbm, dst_hbm, hs_hbm, hd_hbm, acc, idxb):
        cid = lax.axis_index("c")
        sid = lax.axis_index("s")

        def zero(i, carry):
            acc[pl.ds(i * 16, 16)] = jnp.zeros((16,), _F32)
            return carry
        lax.fori_loop(0, n // 16, zero, 0)

        ones = jnp.full((16,), 1.0, _F32)
        base0 = sid * ept

        def winloop(i, carry):
            base = base0 + i * win

            @pl.when(cid == 0)
            def _():
                pltpu.sync_copy(src_hbm.at[pl.ds(base, win)], idxb)

            @pl.when(cid == 1)
            def _():
                pltpu.sync_copy(dst_hbm.at[pl.ds(base, win)], idxb)

            def chunk(j, carry2):
                i16 = idxb[pl.ds(j * 16, 16)]
                plsc.addupdate_scatter(acc, [i16], ones)
                return carry2
            lax.fori_loop(0, win // 16, chunk, 0, unroll=16)
            return carry
        lax.fori_loop(0, nwin, winloop, 0)

        @pl.when(cid == 0)
        def _():
            pltpu.sync_copy(acc, hs_hbm.at[sid])

        @pl.when(cid == 1)
        def _():
            pltpu.sync_copy(acc, hd_hbm.at[sid])

    return k(src, dst)


def _sc_edge_weights(rs, ri, src, dst):
    n = rs.shape[0]
    e = src.shape[0]
    win = 1280
    ept = e // 32
    nwin = ept // win

    @functools.partial(
        pl.kernel,
        out_type=[jax.ShapeDtypeStruct((e,), _F32)] * 2,
        scratch_types=[
            pltpu.VMEM((n,), _F32),
            pltpu.VMEM((n,), _F32),
            pltpu.VMEM((win,), _I32),
            pltpu.VMEM((win,), _I32),
            pltpu.VMEM((win,), _F32),
            pltpu.VMEM((win,), _F32),
            pltpu.SemaphoreType.DMA,
        ],
        **_SC_PARAMS,
    )
    def k(rs_hbm, ri_hbm, src_hbm, dst_hbm, w1_hbm, w2_hbm,
          rs_v, ri_v, srcb, dstb, w1b, w2b, sem):
        cid = lax.axis_index("c")
        sid = lax.axis_index("s")
        wid = cid * 16 + sid
        pltpu.sync_copy(rs_hbm, rs_v)
        pltpu.sync_copy(ri_hbm, ri_v)
        base0 = wid * ept

        def wloop(i, carry):
            base = base0 + i * win
            cp1 = pltpu.async_copy(src_hbm.at[pl.ds(base, win)], srcb, sem)
            cp2 = pltpu.async_copy(dst_hbm.at[pl.ds(base, win)], dstb, sem)
            cp1.wait()
            cp2.wait()

            def chunk(j, carry2):
                sl = pl.ds(j * 16, 16)
                a = plsc.load_gather(rs_v, [srcb[sl]])
                b = plsc.load_gather(ri_v, [dstb[sl]])
                w = -(a * b)
                w1b[sl] = w
                w2b[sl] = w + w
                return carry2
            lax.fori_loop(0, win // 16, chunk, 0, unroll=16)
            pltpu.sync_copy(w1b, w1_hbm.at[pl.ds(base, win)])
            pltpu.sync_copy(w2b, w2_hbm.at[pl.ds(base, win)])
            return carry
        lax.fori_loop(0, nwin, wloop, 0)

    return k(rs, ri, src, dst)


def _sc_adj_pass(x_flat, init_flat, src, dst, w, n, c):
    e = src.shape[0]
    nch = c // 16
    seg = nch * n
    win = 4096
    ehalf = e // 2
    nwin = ehalf // win

    @functools.partial(
        pl.kernel,
        out_type=[jax.ShapeDtypeStruct((c * n,), _F32)] * 2,
        scratch_types=[
            pltpu.VMEM((seg,), _F32),
            pltpu.VMEM((seg,), _F32),
            pltpu.VMEM((win,), _I32),
            pltpu.VMEM((win,), _I32),
            pltpu.VMEM((win,), _F32),
            pltpu.VMEM((win,), _I32),
            pltpu.VMEM((win,), _I32),
            pltpu.VMEM((win,), _F32),
            pltpu.SemaphoreType.DMA,
            pltpu.SemaphoreType.DMA,
        ],
        **_SC_PARAMS,
    )
    def k(x_hbm, init_hbm, src_hbm, dst_hbm, w_hbm, out0_hbm, out1_hbm,
          xrows, acc, srcb0, dstb0, wb0, srcb1, dstb1, wb1, sem0, sem1):
        cid = lax.axis_index("c")
        sid = lax.axis_index("s")
        off = sid * seg
        pltpu.sync_copy(x_hbm.at[pl.ds(off, seg)], xrows)

        @pl.when(cid == 0)
        def _():
            pltpu.sync_copy(init_hbm.at[pl.ds(off, seg)], acc)

        @pl.when(cid == 1)
        def _():
            def zero(i, carry):
                acc[pl.ds(i * 16, 16)] = jnp.zeros((16,), _F32)
                return carry
            lax.fori_loop(0, seg // 16, zero, 0)

        base0 = cid * ehalf

        def fire(i, bufs, sem):
            base = base0 + i * win
            pltpu.async_copy(src_hbm.at[pl.ds(base, win)], bufs[0], sem)
            pltpu.async_copy(dst_hbm.at[pl.ds(base, win)], bufs[1], sem)
            pltpu.async_copy(w_hbm.at[pl.ds(base, win)], bufs[2], sem)

        def drain(i, bufs, sem):
            base = base0 + i * win
            pltpu.make_async_copy(
                src_hbm.at[pl.ds(base, win)], bufs[0], sem).wait()
            pltpu.make_async_copy(
                dst_hbm.at[pl.ds(base, win)], bufs[1], sem).wait()
            pltpu.make_async_copy(
                w_hbm.at[pl.ds(base, win)], bufs[2], sem).wait()

        def process(bufs):
            srcb, dstb, wb = bufs

            def chunk(j, carry2):
                sl = pl.ds(j * 16, 16)
                s16 = srcb[sl]
                d16 = dstb[sl]
                w16 = wb[sl]
                for kk in range(nch):
                    ko = kk * n
                    v = plsc.load_gather(xrows, [s16 + ko])
                    plsc.addupdate_scatter(acc, [d16 + ko], v * w16)
                return carry2
            lax.fori_loop(0, win // 16, chunk, 0, unroll=16)

        bufs0 = (srcb0, dstb0, wb0)
        bufs1 = (srcb1, dstb1, wb1)
        fire(0, bufs0, sem0)

        def winloop(g, carry):
            i0 = 2 * g
            i1 = 2 * g + 1

            @pl.when(i1 < nwin)
            def _():
                fire(i1, bufs1, sem1)
            drain(i0, bufs0, sem0)
            process(bufs0)

            @pl.when(i1 + 1 < nwin)
            def _():
                fire(i1 + 1, bufs0, sem0)

            @pl.when(i1 < nwin)
            def _():
                drain(i1, bufs1, sem1)
                process(bufs1)
            return carry
        lax.fori_loop(0, (nwin + 1) // 2, winloop, 0)

        @pl.when(cid == 0)
        def _():
            pltpu.sync_copy(acc, out0_hbm.at[pl.ds(off, seg)])

        @pl.when(cid == 1)
        def _():
            pltpu.sync_copy(acc, out1_hbm.at[pl.ds(off, seg)])

    o0, o1 = k(x_flat, init_flat, src, dst, w)
    return o0, o1


def _adj(x, init, src, dst, w):
    c, n = x.shape
    o0, o1 = _sc_adj_pass(x.reshape(-1), init.reshape(-1), src, dst, w, n, c)
    return o0.reshape(c, n), o1.reshape(c, n)


def kernel(x_enc0, x_enc1, x_enc2, x_enc3, x_enc4, x_enc5,
           ei1, ei2, ei3, ei4, ei5, params):
    encs = [x_enc0, x_enc1, x_enc2, x_enc3, x_enc4, x_enc5]
    eis = [ei1, ei2, ei3, ei4, ei5]

    prevs = [x_enc0[0]]
    for lvl in range(1, 6):
        ei = eis[lvl - 1]
        src = ei[0]
        dst = ei[1]
        n = ei.shape[1] // 16
        bp = params['block%d' % lvl]
        W1, b1 = bp['W1'], bp['b1']
        W2, b2 = bp['W2'], bp['b2']
        Ws, bs = bp['Ws'], bp['bs']
        c = W1.shape[2]
        enc = encs[lvl][0]
        c_pr = prevs[0].shape[0]

        hs_p, hd_p = _sc_degree_hist(src, dst, n)
        rs, ri = _rsqrt_deg(hs_p, hd_p)
        w1e, w2e = _sc_edge_weights(rs, ri, src, dst)

        def t(wmat):
            return jnp.transpose(wmat)

        zc = jnp.zeros((c, 1), _F32)
        w10m2 = W1[0] - W1[2]
        wts = [
            (t(w10m2[:c_pr]), t(w10m2[c_pr:]), b1.reshape(c, 1)),
            (t(W1[1][:c_pr]), t(W1[1][c_pr:]), zc),
            (t(W1[2][:c_pr]), t(W1[2][c_pr:]), zc),
            (t(Ws[0][:c_pr]), t(Ws[0][c_pr:]), bs.reshape(c, 1)),
        ]
        p0, p1, p2, s = _matmul1(prevs, enc, wts, relu_prev=(lvl != 1))

        r1a, r1b = _adj(p2, p1, src, dst, w2e)
        r1 = _comb2(r1a, r1b)
        ha, hb = _adj(r1, p0, src, dst, w1e)

        q0, q1, q2 = _matmul2(ha, hb, s, t(W2[0] - W2[2]), t(W2[1]), t(W2[2]),
                              b2.reshape(c, 1))

        r2a, r2b = _adj(q2, q1, src, dst, w2e)
        r2 = _comb2(r2a, r2b)
        oa, ob = _adj(r2, q0, src, dst, w1e)
        prevs = [oa, ob]

    out = _final_head(prevs[0], prevs[1], jnp.transpose(params['conv_W'][0]))
    return out.reshape(1, 32, 1)


# speedup vs baseline: 1.0061x; 1.0061x over previous
import functools

import jax
import jax.numpy as jnp
from jax import lax
from jax.experimental import pallas as pl
from jax.experimental.pallas import tpu as pltpu
from jax.experimental.pallas import tpu_sc as plsc

_F32 = jnp.float32
_I32 = jnp.int32


_BM = 512


def _interleave_mat(bm):
    col = jax.lax.broadcasted_iota(_I32, (bm // 2, bm), 1)
    row = jax.lax.broadcasted_iota(_I32, (bm // 2, bm), 0)
    return (col // 2 == row).astype(_F32)


def _matmul1(prevs, enc, wts, relu_prev):
    c_e, n = enc.shape
    c_pr = prevs[0].shape[0]
    np_ = len(prevs)
    nw = len(wts)
    rmat = _interleave_mat(_BM)

    def body(*refs):
        prefs = refs[:np_]
        enc_ref = refs[np_]
        rref = refs[np_ + 1]
        wrefs = refs[np_ + 2:np_ + 2 + 3 * nw]
        orefs = refs[np_ + 2 + 3 * nw:]
        p = prefs[0][...]
        for pr in prefs[1:]:
            p = p + pr[...]
        if relu_prev:
            p = jnp.maximum(p, 0.0)
        e = enc_ref[...]
        r = rref[...]
        for j in range(nw):
            wt = wrefs[3 * j][...]
            wb = wrefs[3 * j + 1][...]
            bb = wrefs[3 * j + 2][...]
            y = jnp.dot(jnp.dot(wt, p, preferred_element_type=_F32), r,
                        preferred_element_type=_F32)
            orefs[j][...] = y + jnp.dot(wb, e, preferred_element_type=_F32) + bb

    in_specs = [pl.BlockSpec((c_pr, _BM // 2), lambda i: (0, i))] * np_
    in_specs.append(pl.BlockSpec((c_e, _BM), lambda i: (0, i)))
    in_specs.append(pl.BlockSpec((_BM // 2, _BM), lambda i: (0, 0)))
    out_specs = []
    out_shape = []
    for (wt, wb, bb) in wts:
        c = wt.shape[0]
        in_specs.append(pl.BlockSpec(wt.shape, lambda i: (0, 0)))
        in_specs.append(pl.BlockSpec(wb.shape, lambda i: (0, 0)))
        in_specs.append(pl.BlockSpec(bb.shape, lambda i: (0, 0)))
        out_specs.append(pl.BlockSpec((c, _BM), lambda i: (0, i)))
        out_shape.append(jax.ShapeDtypeStruct((c, n), _F32))

    args = list(prevs) + [enc, rmat]
    for (wt, wb, bb) in wts:
        args += [wt, wb, bb]
    return pl.pallas_call(
        body, grid=(n // _BM,), in_specs=in_specs, out_specs=out_specs,
        out_shape=out_shape)(*args)


def _matmul2(ha, hb, s, w0t, w1t, w2t, b2):
    c, n = ha.shape

    def body(ha_ref, hb_ref, s_ref, w0_ref, w1_ref, w2_ref, b2_ref,
             o0, o1, o2):
        h = jnp.maximum(ha_ref[...] + hb_ref[...], 0.0)
        o0[...] = (jnp.dot(w0_ref[...], h, preferred_element_type=_F32)
                   + b2_ref[...] + s_ref[...])
        o1[...] = jnp.dot(w1_ref[...], h, preferred_element_type=_F32)
        o2[...] = jnp.dot(w2_ref[...], h, preferred_element_type=_F32)

    bspec = pl.BlockSpec((c, _BM), lambda i: (0, i))
    wspec = pl.BlockSpec((c, c), lambda i: (0, 0))
    return pl.pallas_call(
        body, grid=(n // _BM,),
        in_specs=[bspec, bspec, bspec, wspec, wspec, wspec,
                  pl.BlockSpec((c, 1), lambda i: (0, 0))],
        out_specs=[bspec, bspec, bspec],
        out_shape=[jax.ShapeDtypeStruct((c, n), _F32)] * 3,
    )(ha, hb, s, w0t, w1t, w2t, b2)


def _comb2(a, b):
    c, n = a.shape

    def body(a_ref, b_ref, o_ref):
        o_ref[...] = a_ref[...] + b_ref[...]

    bspec = pl.BlockSpec((c, _BM), lambda i: (0, i))
    return pl.pallas_call(
        body, grid=(n // _BM,), in_specs=[bspec, bspec], out_specs=bspec,
        out_shape=jax.ShapeDtypeStruct((c, n), _F32))(a, b)


def _rsqrt_deg(hs_p, hd_p):
    n = hs_p.shape[1]

    def body(hs_ref, hd_ref, rs_ref, ri_ref):
        ds = jnp.maximum(jnp.sum(hs_ref[...], axis=0), 1.0)
        di = jnp.maximum(jnp.sum(hd_ref[...], axis=0), 1.0)
        rs_ref[...] = lax.rsqrt(ds)
        ri_ref[...] = lax.rsqrt(di)

    return pl.pallas_call(
        body,
        in_specs=[pl.BlockSpec((16, n), lambda: (0, 0))] * 2,
        out_specs=[pl.BlockSpec((n,), lambda: (0,))] * 2,
        out_shape=[jax.ShapeDtypeStruct((n,), _F32)] * 2,
    )(hs_p, hd_p)


def _final_head(oa, ob, wc_t):
    n = oa.shape[1]

    def body(oa_ref, ob_ref, w_ref, out_ref):
        o = jnp.maximum(oa_ref[...] + ob_ref[...], 0.0)
        pooled = jnp.mean(o, axis=1, keepdims=True)
        logits = jnp.dot(w_ref[...], pooled, preferred_element_type=_F32)
        m = jnp.max(logits, axis=0, keepdims=True)
        z = logits - m
        lse = jnp.log(jnp.sum(jnp.exp(z), axis=0, keepdims=True))
        out_ref[...] = z - lse

    return pl.pallas_call(
        body,
        in_specs=[pl.BlockSpec((16, n), lambda: (0, 0)),
                  pl.BlockSpec((16, n), lambda: (0, 0)),
                  pl.BlockSpec((32, 16), lambda: (0, 0))],
        out_specs=pl.BlockSpec((32, 1), lambda: (0, 0)),
        out_shape=jax.ShapeDtypeStruct((32, 1), _F32),
    )(oa, ob, wc_t)


_MESH = dict(core_axis_name="c", subcore_axis_name="s")
_SC_PARAMS = dict(
    mesh=plsc.VectorSubcoreMesh(**_MESH),
    compiler_params=pltpu.CompilerParams(needs_layout_passes=False),
)


def _sc_degree_hist(src, dst, n):
    e = src.shape[0]
    ept = e // 16
    win = 2560
    nwin = ept // win

    @functools.partial(
        pl.kernel,
        out_type=[jax.ShapeDtypeStruct((16, n), _F32)] * 2,
        scratch_types=[
            pltpu.VMEM((n,), _F32),
            pltpu.VMEM((win,), _I32),
        ],
        **_SC_PARAMS,
    )
    def k(src_hbm, dst_hbm, hs_hbm, hd_hbm, acc, idxb):
        cid = lax.axis_index("c")
        sid = lax.axis_index("s")

        def zero(i, carry):
            acc[pl.ds(i * 16, 16)] = jnp.zeros((16,), _F32)
            return carry
        lax.fori_loop(0, n // 16, zero, 0)

        ones = jnp.full((16,), 1.0, _F32)
        base0 = sid * ept

        def winloop(i, carry):
            base = base0 + i * win

            @pl.when(cid == 0)
            def _():
                pltpu.sync_copy(src_hbm.at[pl.ds(base, win)], idxb)

            @pl.when(cid == 1)
            def _():
                pltpu.sync_copy(dst_hbm.at[pl.ds(base, win)], idxb)

            def chunk(j, carry2):
                i16 = idxb[pl.ds(j * 16, 16)]
                plsc.addupdate_scatter(acc, [i16], ones)
                return carry2
            lax.fori_loop(0, win // 16, chunk, 0, unroll=8)
            return carry
        lax.fori_loop(0, nwin, winloop, 0)

        @pl.when(cid == 0)
        def _():
            pltpu.sync_copy(acc, hs_hbm.at[sid])

        @pl.when(cid == 1)
        def _():
            pltpu.sync_copy(acc, hd_hbm.at[sid])

    return k(src, dst)


def _sc_edge_weights(rs, ri, src, dst):
    n = rs.shape[0]
    e = src.shape[0]
    win = 1280
    ept = e // 32
    nwin = ept // win

    @functools.partial(
        pl.kernel,
        out_type=[jax.ShapeDtypeStruct((e,), _F32)] * 2,
        scratch_types=[
            pltpu.VMEM((n,), _F32),
            pltpu.VMEM((n,), _F32),
            pltpu.VMEM((win,), _I32),
            pltpu.VMEM((win,), _I32),
            pltpu.VMEM((win,), _F32),
            pltpu.VMEM((win,), _F32),
            pltpu.SemaphoreType.DMA,
        ],
        **_SC_PARAMS,
    )
    def k(rs_hbm, ri_hbm, src_hbm, dst_hbm, w1_hbm, w2_hbm,
          rs_v, ri_v, srcb, dstb, w1b, w2b, sem):
        cid = lax.axis_index("c")
        sid = lax.axis_index("s")
        wid = cid * 16 + sid
        pltpu.sync_copy(rs_hbm, rs_v)
        pltpu.sync_copy(ri_hbm, ri_v)
        base0 = wid * ept

        def wloop(i, carry):
            base = base0 + i * win
            cp1 = pltpu.async_copy(src_hbm.at[pl.ds(base, win)], srcb, sem)
            cp2 = pltpu.async_copy(dst_hbm.at[pl.ds(base, win)], dstb, sem)
            cp1.wait()
            cp2.wait()

            def chunk(j, carry2):
                sl = pl.ds(j * 16, 16)
                a = plsc.load_gather(rs_v, [srcb[sl]])
                b = plsc.load_gather(ri_v, [dstb[sl]])
                w = -(a * b)
                w1b[sl] = w
                w2b[sl] = w + w
                return carry2
            lax.fori_loop(0, win // 16, chunk, 0, unroll=8)
            pltpu.sync_copy(w1b, w1_hbm.at[pl.ds(base, win)])
            pltpu.sync_copy(w2b, w2_hbm.at[pl.ds(base, win)])
            return carry
        lax.fori_loop(0, nwin, wloop, 0)

    return k(rs, ri, src, dst)


def _sc_adj_pass(x_flat, init_flat, src, dst, w, n, c):
    e = src.shape[0]
    nch = c // 16
    seg = nch * n
    win = 4096
    ehalf = e // 2
    nwin = ehalf // win

    @functools.partial(
        pl.kernel,
        out_type=[jax.ShapeDtypeStruct((c * n,), _F32)] * 2,
        scratch_types=[
            pltpu.VMEM((seg,), _F32),
            pltpu.VMEM((seg,), _F32),
            pltpu.VMEM((win,), _I32),
            pltpu.VMEM((win,), _I32),
            pltpu.VMEM((win,), _F32),
            pltpu.VMEM((win,), _I32),
            pltpu.VMEM((win,), _I32),
            pltpu.VMEM((win,), _F32),
            pltpu.SemaphoreType.DMA,
            pltpu.SemaphoreType.DMA,
        ],
        **_SC_PARAMS,
    )
    def k(x_hbm, init_hbm, src_hbm, dst_hbm, w_hbm, out0_hbm, out1_hbm,
          xrows, acc, srcb0, dstb0, wb0, srcb1, dstb1, wb1, sem0, sem1):
        cid = lax.axis_index("c")
        sid = lax.axis_index("s")
        off = sid * seg
        pltpu.sync_copy(x_hbm.at[pl.ds(off, seg)], xrows)

        @pl.when(cid == 0)
        def _():
            pltpu.sync_copy(init_hbm.at[pl.ds(off, seg)], acc)

        @pl.when(cid == 1)
        def _():
            def zero(i, carry):
                acc[pl.ds(i * 16, 16)] = jnp.zeros((16,), _F32)
                return carry
            lax.fori_loop(0, seg // 16, zero, 0)

        base0 = cid * ehalf

        def fire(i, bufs, sem):
            base = base0 + i * win
            pltpu.async_copy(src_hbm.at[pl.ds(base, win)], bufs[0], sem)
            pltpu.async_copy(dst_hbm.at[pl.ds(base, win)], bufs[1], sem)
            pltpu.async_copy(w_hbm.at[pl.ds(base, win)], bufs[2], sem)

        def drain(i, bufs, sem):
            base = base0 + i * win
            pltpu.make_async_copy(
                src_hbm.at[pl.ds(base, win)], bufs[0], sem).wait()
            pltpu.make_async_copy(
                dst_hbm.at[pl.ds(base, win)], bufs[1], sem).wait()
            pltpu.make_async_copy(
                w_hbm.at[pl.ds(base, win)], bufs[2], sem).wait()

        def process(bufs):
            srcb, dstb, wb = bufs

            def chunk(j, carry2):
                sl = pl.ds(j * 16, 16)
                s16 = srcb[sl]
                d16 = dstb[sl]
                w16 = wb[sl]
                for kk in range(nch):
                    ko = kk * n
                    v = plsc.load_gather(xrows, [s16 + ko])
                    plsc.addupdate_scatter(acc, [d16 + ko], v * w16)
                return carry2
            lax.fori_loop(0, win // 16, chunk, 0, unroll=8)

        bufs0 = (srcb0, dstb0, wb0)
        bufs1 = (srcb1, dstb1, wb1)
        fire(0, bufs0, sem0)

        def winloop(g, carry):
            i0 = 2 * g
            i1 = 2 * g + 1

            @pl.when(i1 < nwin)
            def _():
                fire(i1, bufs1, sem1)
            drain(i0, bufs0, sem0)
            process(bufs0)

            @pl.when(i1 + 1 < nwin)
            def _():
                fire(i1 + 1, bufs0, sem0)

            @pl.when(i1 < nwin)
            def _():
                drain(i1, bufs1, sem1)
                process(bufs1)
            return carry
        lax.fori_loop(0, (nwin + 1) // 2, winloop, 0)

        @pl.when(cid == 0)
        def _():
            pltpu.sync_copy(acc, out0_hbm.at[pl.ds(off, seg)])

        @pl.when(cid == 1)
        def _():
            pltpu.sync_copy(acc, out1_hbm.at[pl.ds(off, seg)])

    o0, o1 = k(x_flat, init_flat, src, dst, w)
    return o0, o1


def _adj(x, init, src, dst, w):
    c, n = x.shape
    o0, o1 = _sc_adj_pass(x.reshape(-1), init.reshape(-1), src, dst, w, n, c)
    return o0.reshape(c, n), o1.reshape(c, n)


def kernel(x_enc0, x_enc1, x_enc2, x_enc3, x_enc4, x_enc5,
           ei1, ei2, ei3, ei4, ei5, params):
    encs = [x_enc0, x_enc1, x_enc2, x_enc3, x_enc4, x_enc5]
    eis = [ei1, ei2, ei3, ei4, ei5]

    prevs = [x_enc0[0]]
    for lvl in range(1, 6):
        ei = eis[lvl - 1]
        src = ei[0]
        dst = ei[1]
        n = ei.shape[1] // 16
        bp = params['block%d' % lvl]
        W1, b1 = bp['W1'], bp['b1']
        W2, b2 = bp['W2'], bp['b2']
        Ws, bs = bp['Ws'], bp['bs']
        c = W1.shape[2]
        enc = encs[lvl][0]
        c_pr = prevs[0].shape[0]

        hs_p, hd_p = _sc_degree_hist(src, dst, n)
        rs, ri = _rsqrt_deg(hs_p, hd_p)
        w1e, w2e = _sc_edge_weights(rs, ri, src, dst)

        def t(wmat):
            return jnp.transpose(wmat)

        zc = jnp.zeros((c, 1), _F32)
        w10m2 = W1[0] - W1[2]
        wts = [
            (t(w10m2[:c_pr]), t(w10m2[c_pr:]), b1.reshape(c, 1)),
            (t(W1[1][:c_pr]), t(W1[1][c_pr:]), zc),
            (t(W1[2][:c_pr]), t(W1[2][c_pr:]), zc),
            (t(Ws[0][:c_pr]), t(Ws[0][c_pr:]), bs.reshape(c, 1)),
        ]
        p0, p1, p2, s = _matmul1(prevs, enc, wts, relu_prev=(lvl != 1))

        r1a, r1b = _adj(p2, p1, src, dst, w2e)
        r1 = _comb2(r1a, r1b)
        ha, hb = _adj(r1, p0, src, dst, w1e)

        q0, q1, q2 = _matmul2(ha, hb, s, t(W2[0] - W2[2]), t(W2[1]), t(W2[2]),
                              b2.reshape(c, 1))

        r2a, r2b = _adj(q2, q1, src, dst, w2e)
        r2 = _comb2(r2a, r2b)
        oa, ob = _adj(r2, q0, src, dst, w1e)
        prevs = [oa, ob]

    out = _final_head(prevs[0], prevs[1], jnp.transpose(params['conv_W'][0]))
    return out.reshape(1, 32, 1)
